# Initial kernel scaffold; baseline (speedup 1.0000x reference)
import jax
import jax.numpy as jnp
from jax.experimental import pallas as pl


def kernel(n, region, cdr_type, chain, interface, W_region, W_cdr, W_chain, W_iface, W_proj, b_proj):
    raise NotImplementedError("write your pallas kernel here")


# speedup vs baseline: 9.9052x; 9.9052x over previous
import functools

import jax
import jax.numpy as jnp
from jax import lax
from jax.experimental import pallas as pl
from jax.experimental.pallas import tpu as pltpu
from jax.experimental.pallas import tpu_sc as plsc

_LANES = 16


def _vsplat(vec, lane):
    idx = jnp.full((_LANES, 1), lane, jnp.int32)
    dnums = lax.GatherDimensionNumbers(
        offset_dims=(), collapsed_slice_dims=(0,), start_index_map=(0,))
    return lax.gather(vec, idx, dnums, (1,),
                      mode=lax.GatherScatterMode.PROMISE_IN_BOUNDS)
_CHUNK = 256
_IDXW = 128
_NW = 32


@functools.lru_cache(maxsize=None)
def _build_sc_call(T, D, nreg, ncdr, nch, nif, ncont):
    tokens_per_worker = T // _NW
    n_chunks = tokens_per_worker // _CHUNK
    cont_base = D - ncont
    mesh = plsc.VectorSubcoreMesh(core_axis_name="c", subcore_axis_name="s")

    @functools.partial(
        pl.kernel,
        mesh=mesh,
        out_type=jax.ShapeDtypeStruct((T, D), jnp.float32),
        scratch_types=[
            pltpu.VMEM((_CHUNK,), jnp.int32),
            pltpu.VMEM((_CHUNK,), jnp.int32),
            pltpu.VMEM((_CHUNK,), jnp.int32),
            pltpu.VMEM((_CHUNK,), jnp.int32),
            pltpu.VMEM((_CHUNK,), jnp.float32),
            pltpu.VMEM((_IDXW,), jnp.int32),
            pltpu.VMEM((_IDXW,), jnp.int32),
            pltpu.VMEM((_CHUNK, D), jnp.float32),
            pltpu.VMEM((ncont,), jnp.float32),
            pltpu.SemaphoreType.DMA,
        ],
    )
    def sc_call(n_h, reg_h, cdr_h, ch_h, if_h, tab_h, w_h, out_h,
                reg_v, cdr_v, ch_v, if_v, n_v, cidx0_v, cidx1_v, rows_v,
                w_v, gsem):
        wid = lax.axis_index("s") * 2 + lax.axis_index("c")
        base = wid * tokens_per_worker

        pltpu.sync_copy(w_h, w_v)
        w_slices = [w_v[pl.ds(k * _LANES, _LANES)] for k in range(ncont // _LANES)]

        def chunk_body(g, carry):
            off = base + g * _CHUNK
            pltpu.sync_copy(reg_h.at[pl.ds(off, _CHUNK)], reg_v)
            pltpu.sync_copy(cdr_h.at[pl.ds(off, _CHUNK)], cdr_v)
            pltpu.sync_copy(ch_h.at[pl.ds(off, _CHUNK)], ch_v)
            pltpu.sync_copy(if_h.at[pl.ds(off, _CHUNK)], if_v)
            pltpu.sync_copy(n_h.at[pl.ds(off, _CHUNK)], n_v)

            for i in range(_CHUNK // _LANES):
                s = pl.ds(i * _LANES, _LANES)
                cidx = ((reg_v[s] * ncdr + cdr_v[s]) * nch + ch_v[s]) * nif + if_v[s]
                dst = cidx0_v if i < _IDXW // _LANES else cidx1_v
                dst[pl.ds((i % (_IDXW // _LANES)) * _LANES, _LANES)] = cidx

            h0 = pltpu.async_copy(tab_h.at[cidx0_v], rows_v.at[pl.ds(0, _IDXW)], gsem)
            h1 = pltpu.async_copy(tab_h.at[cidx1_v], rows_v.at[pl.ds(_IDXW, _IDXW)], gsem)
            h0.wait()
            h1.wait()

            for gi in range(_CHUNK // _LANES):
                n16 = n_v[pl.ds(gi * _LANES, _LANES)]
                for tl in range(_LANES):
                    sp = _vsplat(n16, tl)
                    t = gi * _LANES + tl
                    for k in range(ncont // _LANES):
                        plsc.addupdate(
                            rows_v.at[t, pl.ds(cont_base + k * _LANES, _LANES)],
                            sp * w_slices[k])

            pltpu.sync_copy(rows_v, out_h.at[pl.ds(off, _CHUNK)])
            return carry

        lax.fori_loop(0, n_chunks, chunk_body, 0)

    return sc_call


def _combined_table(W_region, W_cdr, W_chain, W_iface, b_proj):
    nreg, ncdr, nch, nif = (W_region.shape[0], W_cdr.shape[0],
                            W_chain.shape[0], W_iface.shape[0])
    rows = nreg * ncdr * nch * nif
    ridx = jnp.arange(rows)
    f = ridx % nif
    ch = (ridx // nif) % nch
    c = (ridx // (nif * nch)) % ncdr
    r = ridx // (nif * nch * ncdr)
    bias = jnp.broadcast_to(b_proj[None, :], (rows, b_proj.shape[0]))
    return jnp.concatenate(
        [W_region[r], W_cdr[c], W_chain[ch], W_iface[f], bias], axis=1)


def kernel(n, region, cdr_type, chain, interface,
           W_region, W_cdr, W_chain, W_iface, W_proj, b_proj):
    B, L = n.shape
    ncont = W_proj.shape[0]
    D = (W_region.shape[1] + W_cdr.shape[1] + W_chain.shape[1]
         + W_iface.shape[1] + ncont)
    T = B * L
    tab = _combined_table(W_region, W_cdr, W_chain, W_iface, b_proj)
    call = _build_sc_call(T, D, W_region.shape[0], W_cdr.shape[0],
                          W_chain.shape[0], W_iface.shape[0], ncont)
    out = call(n.reshape(T), region.reshape(T), cdr_type.reshape(T),
               chain.reshape(T), interface.reshape(T), tab,
               W_proj.reshape(ncont))
    return out.reshape(B, L, D)
